# baseline (speedup 1.0000x reference)
import functools

import jax
import jax.numpy as jnp
from jax import lax
from jax.experimental import pallas as pl
from jax.experimental.pallas import tpu as pltpu
from jax.experimental.pallas import tpu_sc as plsc

T, B, C, H, W = 4, 32, 768, 14, 14
HW = H * W
E = 64
TOP_K = 8
N_TOK = T * B

_NC = 2
_NS = 16
_NW = _NC * _NS
_ROWS = N_TOK // _NW


_HWB = 49
_GRID = HW // _HWB


def _stage1_body(x_ref, w_ref, g_ref, bb_ref, out_ref, out_t_ref, p_scr, q_scr):
    i = pl.program_id(0)
    xb = x_ref[...]
    v = jnp.zeros((_HWB, B, C), jnp.float32)
    spikes = []
    for t in range(T):
        v = v + (xb[t] - v) / 2.0
        fire = v >= 1.0
        spikes.append(fire.astype(jnp.float32))
        v = jnp.where(fire, 0.0, v)
    s2 = jnp.stack(spikes).reshape(T * _HWB * B, C)
    d = lax.dot_general(s2, w_ref[...], (((1,), (1,)), ((), ())),
                        precision=lax.Precision.DEFAULT,
                        preferred_element_type=jnp.float32)
    d4 = d.reshape(T, _HWB, B, E)
    p_par = jnp.sum(d4, axis=1)
    q_par = jnp.sum(d * d, axis=0)[None, :]

    @pl.when(i == 0)
    def _():
        p_scr[...] = jnp.zeros_like(p_scr)
        q_scr[...] = jnp.zeros_like(q_scr)

    p_scr[...] += p_par
    q_scr[...] += q_par

    @pl.when(i == pl.num_programs(0) - 1)
    def _():
        P = p_scr[...].reshape(N_TOK, E)
        n_elems = float(N_TOK * HW)
        mu = jnp.sum(P, axis=0, keepdims=True) / n_elems
        var = q_scr[...] / n_elems - mu * mu
        scale = g_ref[...] / jnp.sqrt(var + 1e-5)
        logits = (P / float(HW) - mu) * scale + bb_ref[...]
        out_ref[...] = logits
        out_t_ref[...] = logits.T


def _stage1(xs, conv_wt, g2, b2):
    return pl.pallas_call(
        _stage1_body,
        grid=(_GRID,),
        in_specs=[
            pl.BlockSpec((T, _HWB, B, C), lambda i: (0, i, 0, 0)),
            pl.BlockSpec((E, C), lambda i: (0, 0)),
            pl.BlockSpec((1, E), lambda i: (0, 0)),
            pl.BlockSpec((1, E), lambda i: (0, 0)),
        ],
        out_specs=[pl.BlockSpec((N_TOK, E), lambda i: (0, 0)),
                   pl.BlockSpec((E, N_TOK), lambda i: (0, 0))],
        out_shape=[jax.ShapeDtypeStruct((N_TOK, E), jnp.float32),
                   jax.ShapeDtypeStruct((E, N_TOK), jnp.float32)],
        scratch_shapes=[pltpu.VMEM((T, B, E), jnp.float32),
                        pltpu.VMEM((1, E), jnp.float32)],
    )(xs, conv_wt, g2, b2)


@functools.partial(
    pl.kernel,
    mesh=plsc.VectorSubcoreMesh(core_axis_name="c", subcore_axis_name="s",
                                num_cores=_NC),
    out_type=(jax.ShapeDtypeStruct((N_TOK * TOP_K,), jnp.float32),
              jax.ShapeDtypeStruct((N_TOK * TOP_K,), jnp.int32)),
    scratch_types=[pltpu.VMEM((_ROWS * E,), jnp.float32),
                   pltpu.VMEM((_ROWS * TOP_K,), jnp.float32),
                   pltpu.VMEM((_ROWS * TOP_K,), jnp.int32)],
)
def _router(logits_hbm, w_out, i_out, rows_v, wstage, istage):
    wid = lax.axis_index("s") * _NC + lax.axis_index("c")
    base = wid * _ROWS
    pltpu.sync_copy(logits_hbm.at[pl.ds(base * E, _ROWS * E)], rows_v)
    lane = lax.iota(jnp.int32, 16)

    def permute(vec, idx):
        return lax.gather(
            vec, idx[:, None],
            lax.GatherDimensionNumbers(offset_dims=(),
                                       collapsed_slice_dims=(0,),
                                       start_index_map=(0,)),
            slice_sizes=(1,),
            mode=lax.GatherScatterMode.PROMISE_IN_BOUNDS)

    def allreduce(vec, op):
        for sh in (8, 4, 2, 1):
            vec = op(vec, permute(vec, jnp.bitwise_xor(lane, sh)))
        return vec

    nchunk = E // 16
    npair = (_ROWS * TOP_K) // 16
    wv = [jnp.zeros((16,), jnp.float32) for _ in range(npair)]
    iv = [jnp.zeros((16,), jnp.int32) for _ in range(npair)]
    big = jnp.full((16,), 2147483647, jnp.int32)
    for r in range(_ROWS):
        p, off = r // 2, TOP_K * (r % 2)
        ch = [rows_v[pl.ds(r * E + 16 * j, 16)] for j in range(nchunk)]
        m = allreduce(jnp.maximum(jnp.maximum(ch[0], ch[1]),
                                  jnp.maximum(ch[2], ch[3])), jnp.maximum)
        sm = [jnp.exp(c - m) for c in ch]
        tot = allreduce(sm[0] + sm[1] + sm[2] + sm[3], jnp.add)
        inv = 1.0 / tot
        sm = [s * inv for s in sm]
        wtmp = jnp.zeros((16,), jnp.float32)
        itmp = jnp.zeros((16,), jnp.int32)
        topsum = jnp.zeros((16,), jnp.float32)
        for k in range(TOP_K):
            mx = allreduce(jnp.maximum(jnp.maximum(sm[0], sm[1]),
                                       jnp.maximum(sm[2], sm[3])), jnp.maximum)
            cand = big
            for j in range(nchunk):
                cand = jnp.minimum(cand, jnp.where(sm[j] == mx,
                                                   lane + 16 * j, big))
            cand = allreduce(cand, jnp.minimum)
            wtmp = jnp.where(lane == off + k, mx, wtmp)
            itmp = jnp.where(lane == off + k, cand, itmp)
            topsum = topsum + mx
            for j in range(nchunk):
                sm[j] = jnp.where(lane + 16 * j == cand, jnp.float32(-1.0),
                                  sm[j])
        wv[p] = wv[p] + wtmp * (1.0 / topsum)
        iv[p] = iv[p] + itmp
    for p in range(npair):
        wstage[pl.ds(16 * p, 16)] = wv[p]
        istage[pl.ds(16 * p, 16)] = iv[p]
    pltpu.sync_copy(wstage, w_out.at[pl.ds(base * TOP_K, _ROWS * TOP_K)])
    pltpu.sync_copy(istage, i_out.at[pl.ds(base * TOP_K, _ROWS * TOP_K)])


def kernel(x, conv_w, conv_b, bn_gamma, bn_beta):
    del conv_b
    xs = x.transpose(0, 3, 4, 1, 2).reshape(T, HW, B, C)
    logits, logits_t = _stage1(xs, conv_w, bn_gamma.reshape(1, E),
                               bn_beta.reshape(1, E))
    tkw, tki = _router(logits.reshape(-1))
    return (tkw.reshape(N_TOK, TOP_K), tki.reshape(N_TOK, TOP_K), logits_t.T)


# speedup vs baseline: 1.0087x; 1.0087x over previous
import functools

import jax
import jax.numpy as jnp
from jax import lax
from jax.experimental import pallas as pl
from jax.experimental.pallas import tpu as pltpu
from jax.experimental.pallas import tpu_sc as plsc

T, B, C, H, W = 4, 32, 768, 14, 14
HW = H * W
E = 64
TOP_K = 8
N_TOK = T * B

_NC = 2
_NS = 16
_NW = _NC * _NS
_ROWS = N_TOK // _NW


_HWB = 28
_GRID = HW // _HWB


def _stage1_body(x_ref, w_ref, g_ref, bb_ref, out_ref, out_t_ref, p_scr, q_scr):
    i = pl.program_id(0)
    xb = x_ref[...]
    v = jnp.zeros((_HWB, B, C), jnp.float32)
    spikes = []
    for t in range(T):
        v = v + (xb[t] - v) / 2.0
        fire = v >= 1.0
        spikes.append(fire.astype(jnp.float32))
        v = jnp.where(fire, 0.0, v)
    s2 = jnp.stack(spikes).reshape(T * _HWB * B, C)
    d = lax.dot_general(s2, w_ref[...], (((1,), (1,)), ((), ())),
                        precision=lax.Precision.DEFAULT,
                        preferred_element_type=jnp.float32)
    d4 = d.reshape(T, _HWB, B, E)
    p_par = jnp.sum(d4, axis=1)
    q_par = jnp.sum(d * d, axis=0)[None, :]

    @pl.when(i == 0)
    def _():
        p_scr[...] = jnp.zeros_like(p_scr)
        q_scr[...] = jnp.zeros_like(q_scr)

    p_scr[...] += p_par
    q_scr[...] += q_par

    @pl.when(i == pl.num_programs(0) - 1)
    def _():
        P = p_scr[...].reshape(N_TOK, E)
        n_elems = float(N_TOK * HW)
        mu = jnp.sum(P, axis=0, keepdims=True) / n_elems
        var = q_scr[...] / n_elems - mu * mu
        scale = g_ref[...] / jnp.sqrt(var + 1e-5)
        logits = (P / float(HW) - mu) * scale + bb_ref[...]
        out_ref[...] = logits
        out_t_ref[...] = logits.T


def _stage1(xs, conv_wt, g2, b2):
    return pl.pallas_call(
        _stage1_body,
        grid=(_GRID,),
        in_specs=[
            pl.BlockSpec((T, _HWB, B, C), lambda i: (0, i, 0, 0)),
            pl.BlockSpec((E, C), lambda i: (0, 0)),
            pl.BlockSpec((1, E), lambda i: (0, 0)),
            pl.BlockSpec((1, E), lambda i: (0, 0)),
        ],
        out_specs=[pl.BlockSpec((N_TOK, E), lambda i: (0, 0)),
                   pl.BlockSpec((E, N_TOK), lambda i: (0, 0))],
        out_shape=[jax.ShapeDtypeStruct((N_TOK, E), jnp.float32),
                   jax.ShapeDtypeStruct((E, N_TOK), jnp.float32)],
        scratch_shapes=[pltpu.VMEM((T, B, E), jnp.float32),
                        pltpu.VMEM((1, E), jnp.float32)],
    )(xs, conv_wt, g2, b2)


@functools.partial(
    pl.kernel,
    mesh=plsc.VectorSubcoreMesh(core_axis_name="c", subcore_axis_name="s",
                                num_cores=_NC),
    out_type=(jax.ShapeDtypeStruct((N_TOK * TOP_K,), jnp.float32),
              jax.ShapeDtypeStruct((N_TOK * TOP_K,), jnp.int32)),
    scratch_types=[pltpu.VMEM((_ROWS * E,), jnp.float32),
                   pltpu.VMEM((_ROWS * TOP_K,), jnp.float32),
                   pltpu.VMEM((_ROWS * TOP_K,), jnp.int32)],
)
def _router(logits_hbm, w_out, i_out, rows_v, wstage, istage):
    wid = lax.axis_index("s") * _NC + lax.axis_index("c")
    base = wid * _ROWS
    pltpu.sync_copy(logits_hbm.at[pl.ds(base * E, _ROWS * E)], rows_v)
    lane = lax.iota(jnp.int32, 16)

    def permute(vec, idx):
        return lax.gather(
            vec, idx[:, None],
            lax.GatherDimensionNumbers(offset_dims=(),
                                       collapsed_slice_dims=(0,),
                                       start_index_map=(0,)),
            slice_sizes=(1,),
            mode=lax.GatherScatterMode.PROMISE_IN_BOUNDS)

    def allreduce(vec, op):
        for sh in (8, 4, 2, 1):
            vec = op(vec, permute(vec, jnp.bitwise_xor(lane, sh)))
        return vec

    nchunk = E // 16
    npair = (_ROWS * TOP_K) // 16
    wv = [jnp.zeros((16,), jnp.float32) for _ in range(npair)]
    iv = [jnp.zeros((16,), jnp.int32) for _ in range(npair)]
    big = jnp.full((16,), 2147483647, jnp.int32)
    for r in range(_ROWS):
        p, off = r // 2, TOP_K * (r % 2)
        ch = [rows_v[pl.ds(r * E + 16 * j, 16)] for j in range(nchunk)]
        m = allreduce(jnp.maximum(jnp.maximum(ch[0], ch[1]),
                                  jnp.maximum(ch[2], ch[3])), jnp.maximum)
        sm = [jnp.exp(c - m) for c in ch]
        tot = allreduce(sm[0] + sm[1] + sm[2] + sm[3], jnp.add)
        inv = 1.0 / tot
        sm = [s * inv for s in sm]
        wtmp = jnp.zeros((16,), jnp.float32)
        itmp = jnp.zeros((16,), jnp.int32)
        topsum = jnp.zeros((16,), jnp.float32)
        for k in range(TOP_K):
            mx = allreduce(jnp.maximum(jnp.maximum(sm[0], sm[1]),
                                       jnp.maximum(sm[2], sm[3])), jnp.maximum)
            cand = big
            for j in range(nchunk):
                cand = jnp.minimum(cand, jnp.where(sm[j] == mx,
                                                   lane + 16 * j, big))
            cand = allreduce(cand, jnp.minimum)
            wtmp = jnp.where(lane == off + k, mx, wtmp)
            itmp = jnp.where(lane == off + k, cand, itmp)
            topsum = topsum + mx
            for j in range(nchunk):
                sm[j] = jnp.where(lane + 16 * j == cand, jnp.float32(-1.0),
                                  sm[j])
        wv[p] = wv[p] + wtmp * (1.0 / topsum)
        iv[p] = iv[p] + itmp
    for p in range(npair):
        wstage[pl.ds(16 * p, 16)] = wv[p]
        istage[pl.ds(16 * p, 16)] = iv[p]
    pltpu.sync_copy(wstage, w_out.at[pl.ds(base * TOP_K, _ROWS * TOP_K)])
    pltpu.sync_copy(istage, i_out.at[pl.ds(base * TOP_K, _ROWS * TOP_K)])


def kernel(x, conv_w, conv_b, bn_gamma, bn_beta):
    del conv_b
    xs = x.transpose(0, 3, 4, 1, 2).reshape(T, HW, B, C)
    logits, logits_t = _stage1(xs, conv_w, bn_gamma.reshape(1, E),
                               bn_beta.reshape(1, E))
    tkw, tki = _router(logits.reshape(-1))
    return (tkw.reshape(N_TOK, TOP_K), tki.reshape(N_TOK, TOP_K), logits_t.T)
